# baseline (device time: 120620 ns/iter reference)
import jax
import jax.numpy as jnp
from jax import lax
from jax.experimental import pallas as pl
from jax.experimental.pallas import tpu as pltpu

N_DEV = 4
SQ = 1024
SKV_LOC = 1024
HQ = 8
DH = 128
D = 1024
SCALE = 0.08838834764831843


def kernel(x, Wq, K_ext, V_ext, Wo):
    x2 = x.reshape(SQ, D)
    K2 = K_ext.reshape(SKV_LOC, HQ * DH)
    V2 = V_ext.reshape(SKV_LOC, HQ * DH)

    def body(x_ref, wq_ref, k_ref, v_ref, wo_ref, out_ref,
             acc_comm, l_comm, acc_tot, l_tot,
             acc_send_sems, acc_recv_sems, l_send_sems, l_recv_sems):
        my = lax.axis_index("i")
        left = (my - 1) % N_DEV
        right = (my + 1) % N_DEV

        xb = x_ref[...].astype(jnp.bfloat16)
        wqb = wq_ref[...].astype(jnp.bfloat16)
        q = jnp.dot(xb, wqb, preferred_element_type=jnp.float32)
        q = (q * SCALE).astype(jnp.bfloat16)

        qi = lax.broadcasted_iota(jnp.int32, (SQ, SKV_LOC), 0)
        kj = lax.broadcasted_iota(jnp.int32, (SQ, SKV_LOC), 1) + my * SKV_LOC
        mask = (jnp.abs(qi - kj) <= 128) | (kj < 32) | (qi < 32)

        for h in range(HQ):
            sl = slice(h * DH, (h + 1) * DH)
            kh = k_ref[:, sl].astype(jnp.bfloat16)
            vh = v_ref[:, sl].astype(jnp.bfloat16)
            s = lax.dot_general(
                q[:, sl], kh, (((1,), (1,)), ((), ())),
                preferred_element_type=jnp.float32,
            )
            w = jnp.where(mask, jnp.exp(s), 0.0)
            lh = jnp.sum(w, axis=1, keepdims=True)
            acch = jnp.dot(w.astype(jnp.bfloat16), vh,
                           preferred_element_type=jnp.float32)
            acc_tot[:, sl] = acch
            l_tot[:, h:h + 1] = lh
            acc_comm[0, :, sl] = acch.astype(jnp.bfloat16)
            l_comm[0, :, h:h + 1] = lh

        barrier_sem = pltpu.get_barrier_semaphore()
        for nbr in (left, right):
            pl.semaphore_signal(barrier_sem, inc=1, device_id=(nbr,),
                                device_id_type=pl.DeviceIdType.MESH)
        pl.semaphore_wait(barrier_sem, 2)

        for t in range(N_DEV - 1):
            acc_rdma = pltpu.make_async_remote_copy(
                src_ref=acc_comm.at[t],
                dst_ref=acc_comm.at[t + 1],
                send_sem=acc_send_sems.at[t],
                recv_sem=acc_recv_sems.at[t],
                device_id=(right,),
                device_id_type=pl.DeviceIdType.MESH,
            )
            l_rdma = pltpu.make_async_remote_copy(
                src_ref=l_comm.at[t],
                dst_ref=l_comm.at[t + 1],
                send_sem=l_send_sems.at[t],
                recv_sem=l_recv_sems.at[t],
                device_id=(right,),
                device_id_type=pl.DeviceIdType.MESH,
            )
            acc_rdma.start()
            l_rdma.start()
            acc_rdma.wait()
            l_rdma.wait()
            acc_tot[...] = acc_tot[...] + acc_comm[t + 1].astype(jnp.float32)
            l_tot[...] = l_tot[...] + l_comm[t + 1]

        accf = acc_tot[...]
        lf = l_tot[...]
        ctx = jnp.concatenate(
            [accf[:, h * DH:(h + 1) * DH] / lf[:, h:h + 1] for h in range(HQ)],
            axis=1,
        ).astype(jnp.bfloat16)
        out_ref[...] = jnp.dot(ctx, wo_ref[...].astype(jnp.bfloat16),
                               preferred_element_type=jnp.float32)

    out = pl.pallas_call(
        body,
        out_shape=jax.ShapeDtypeStruct((SQ, D), jnp.float32),
        in_specs=[pl.BlockSpec(memory_space=pltpu.VMEM)] * 5,
        out_specs=pl.BlockSpec(memory_space=pltpu.VMEM),
        scratch_shapes=[
            pltpu.VMEM((N_DEV, SQ, D), jnp.bfloat16),
            pltpu.VMEM((N_DEV, SQ, HQ), jnp.float32),
            pltpu.VMEM((SQ, D), jnp.float32),
            pltpu.VMEM((SQ, HQ), jnp.float32),
            pltpu.SemaphoreType.DMA((N_DEV - 1,)),
            pltpu.SemaphoreType.DMA((N_DEV - 1,)),
            pltpu.SemaphoreType.DMA((N_DEV - 1,)),
            pltpu.SemaphoreType.DMA((N_DEV - 1,)),
        ],
        compiler_params=pltpu.CompilerParams(collective_id=0),
    )(x2, Wq, K2, V2, Wo)
    return out.reshape(1, SQ, D)


# device time: 70835 ns/iter; 1.7028x vs baseline; 1.7028x over previous
import jax
import jax.numpy as jnp
from jax import lax
from jax.experimental import pallas as pl
from jax.experimental.pallas import tpu as pltpu

N_DEV = 4
SQ = 1024
SKV_LOC = 1024
HQ = 8
DH = 128
D = 1024
SCALE = 0.08838834764831843
G = 32
B1 = 896
NB1 = SQ - B1
HALF = SQ // 2


def kernel(x, Wq, K_ext, V_ext, Wo):
    x2 = x.reshape(SQ, D)
    K2 = K_ext.reshape(SKV_LOC, HQ * DH)
    V2 = V_ext.reshape(SKV_LOC, HQ * DH)

    def body(x_ref, wq_ref, k_ref, v_ref, wo_ref, out_ref,
             acc_send, l_send, bcast_acc, bcast_l, strip_acc, strip_l,
             acc_tot, l_tot,
             bcast_send_sems, bcast_recv_sems, fwd_send_sems,
             strip_send_sems, strip_recv_sems):
        my = lax.axis_index("i")

        def rcopy(src, dst, ssem, rsem, dev):
            return pltpu.make_async_remote_copy(
                src_ref=src, dst_ref=dst, send_sem=ssem, recv_sem=rsem,
                device_id=(dev,), device_id_type=pl.DeviceIdType.MESH)

        def wait_recv(dst, rsem):
            rcopy(dst, dst, rsem, rsem, 0).wait_recv()

        def wait_send(src, ssem):
            rcopy(src, src, ssem, ssem, 0).wait_send()

        xb = x_ref[...].astype(jnp.bfloat16)
        wqb = wq_ref[...].astype(jnp.bfloat16)
        q = jnp.dot(xb, wqb, preferred_element_type=jnp.float32)
        q = (q * SCALE).astype(jnp.bfloat16)

        qi = lax.broadcasted_iota(jnp.int32, (SQ, SKV_LOC), 0)
        kj = lax.broadcasted_iota(jnp.int32, (SQ, SKV_LOC), 1) + my * SKV_LOC
        mask = (jnp.abs(qi - kj) <= 128) | (kj < 32) | (qi < 32)

        for h in range(HQ):
            sl = slice(h * DH, (h + 1) * DH)
            kh = k_ref[:, sl].astype(jnp.bfloat16)
            vh = v_ref[:, sl].astype(jnp.bfloat16)
            s = lax.dot_general(
                q[:, sl], kh, (((1,), (1,)), ((), ())),
                preferred_element_type=jnp.float32,
            )
            w = jnp.where(mask, jnp.exp(s), 0.0)
            lh = jnp.sum(w, axis=1, keepdims=True)
            acch = jnp.dot(w.astype(jnp.bfloat16), vh,
                           preferred_element_type=jnp.float32)
            acc_tot[:, sl] = acch
            l_tot[:, h:h + 1] = lh
            acc_send[:, sl] = acch.astype(jnp.bfloat16)
            l_send[:, h:h + 1] = lh

        barrier_sem = pltpu.get_barrier_semaphore()
        for off in (1, 2, 3):
            pl.semaphore_signal(barrier_sem, inc=1,
                                device_id=((my + off) % N_DEV,),
                                device_id_type=pl.DeviceIdType.MESH)
        pl.semaphore_wait(barrier_sem, N_DEV - 1)

        rows_a = pl.ds(0, HALF)
        rows_b = pl.ds(HALF, HALF)
        rows_g = pl.ds(0, G)
        rows_b1 = pl.ds(B1, NB1)
        rows_sg = pl.ds(0, G)
        rows_sb1 = pl.ds(G, NB1)

        @pl.when(my == 0)
        def _():
            rcopy(acc_send.at[rows_a, :], bcast_acc.at[rows_a, :],
                  bcast_send_sems.at[0], bcast_recv_sems.at[0], 1).start()
            rcopy(acc_send.at[rows_b, :], bcast_acc.at[rows_b, :],
                  bcast_send_sems.at[3], bcast_recv_sems.at[1], 3).start()
            rcopy(acc_send.at[rows_b, :], bcast_acc.at[rows_b, :],
                  bcast_send_sems.at[1], bcast_recv_sems.at[1], 1).start()
            rcopy(acc_send.at[rows_a, :], bcast_acc.at[rows_a, :],
                  bcast_send_sems.at[4], bcast_recv_sems.at[0], 3).start()
            rcopy(l_send, bcast_l,
                  bcast_send_sems.at[2], bcast_recv_sems.at[2], 1).start()
            rcopy(l_send, bcast_l,
                  bcast_send_sems.at[5], bcast_recv_sems.at[2], 3).start()

        def send_strips(src_id, dsts, with_b1):
            for d_i, dst in enumerate(dsts):
                rcopy(acc_send.at[rows_g, :], strip_acc.at[src_id, rows_sg, :],
                      strip_send_sems.at[d_i, 0],
                      strip_recv_sems.at[src_id, 0], dst).start()
                rcopy(l_send.at[rows_g, :], strip_l.at[src_id, rows_sg, :],
                      strip_send_sems.at[d_i, 2],
                      strip_recv_sems.at[src_id, 2], dst).start()
                if with_b1:
                    rcopy(acc_send.at[rows_b1, :],
                          strip_acc.at[src_id, rows_sb1, :],
                          strip_send_sems.at[d_i, 1],
                          strip_recv_sems.at[src_id, 1], dst).start()
                    rcopy(l_send.at[rows_b1, :],
                          strip_l.at[src_id, rows_sb1, :],
                          strip_send_sems.at[d_i, 3],
                          strip_recv_sems.at[src_id, 3], dst).start()

        @pl.when(my == 1)
        def _():
            send_strips(1, (0, 2, 3), True)
            wait_recv(bcast_acc.at[rows_a, :], bcast_recv_sems.at[0])
            rcopy(bcast_acc.at[rows_a, :], bcast_acc.at[rows_a, :],
                  fwd_send_sems.at[0], bcast_recv_sems.at[0], 2).start()
            wait_recv(bcast_l, bcast_recv_sems.at[2])
            rcopy(bcast_l, bcast_l,
                  fwd_send_sems.at[1], bcast_recv_sems.at[2], 2).start()
            wait_recv(bcast_acc.at[rows_b, :], bcast_recv_sems.at[1])

        @pl.when(my == 2)
        def _():
            send_strips(2, (0, 1, 3), False)
            wait_recv(bcast_acc.at[rows_a, :], bcast_recv_sems.at[0])
            wait_recv(bcast_acc.at[rows_b, :], bcast_recv_sems.at[1])
            wait_recv(bcast_l, bcast_recv_sems.at[2])

        @pl.when(my == 3)
        def _():
            send_strips(3, (0, 1, 2), False)
            wait_recv(bcast_acc.at[rows_b, :], bcast_recv_sems.at[1])
            rcopy(bcast_acc.at[rows_b, :], bcast_acc.at[rows_b, :],
                  fwd_send_sems.at[0], bcast_recv_sems.at[1], 2).start()
            wait_recv(bcast_acc.at[rows_a, :], bcast_recv_sems.at[0])
            wait_recv(bcast_l, bcast_recv_sems.at[2])

        @pl.when(my != 0)
        def _():
            acc_tot[...] = acc_tot[...] + bcast_acc[...].astype(jnp.float32)
            l_tot[...] = l_tot[...] + bcast_l[...]

        for s_id in (1, 2, 3):
            @pl.when(my != s_id)
            def _(s_id=s_id):
                wait_recv(strip_acc.at[s_id, rows_sg, :],
                          strip_recv_sems.at[s_id, 0])
                wait_recv(strip_l.at[s_id, rows_sg, :],
                          strip_recv_sems.at[s_id, 2])
                acc_tot[rows_g, :] = (
                    acc_tot[rows_g, :]
                    + strip_acc[s_id, rows_sg, :].astype(jnp.float32))
                l_tot[rows_g, :] = l_tot[rows_g, :] + strip_l[s_id, rows_sg, :]
                if s_id == 1:
                    wait_recv(strip_acc.at[s_id, rows_sb1, :],
                              strip_recv_sems.at[s_id, 1])
                    wait_recv(strip_l.at[s_id, rows_sb1, :],
                              strip_recv_sems.at[s_id, 3])
                    acc_tot[rows_b1, :] = (
                        acc_tot[rows_b1, :]
                        + strip_acc[s_id, rows_sb1, :].astype(jnp.float32))
                    l_tot[rows_b1, :] = (
                        l_tot[rows_b1, :] + strip_l[s_id, rows_sb1, :])

        accf = acc_tot[...]
        lf = l_tot[...]
        ctx = jnp.concatenate(
            [accf[:, h * DH:(h + 1) * DH] / lf[:, h:h + 1] for h in range(HQ)],
            axis=1,
        ).astype(jnp.bfloat16)
        out_ref[...] = jnp.dot(ctx, wo_ref[...].astype(jnp.bfloat16),
                               preferred_element_type=jnp.float32)

        @pl.when(my == 0)
        def _():
            for i, rows in ((0, rows_a), (1, rows_b), (3, rows_b), (4, rows_a)):
                wait_send(acc_send.at[rows, :], bcast_send_sems.at[i])
            wait_send(l_send, bcast_send_sems.at[2])
            wait_send(l_send, bcast_send_sems.at[5])

        def drain_strips(with_b1):
            for d_i in range(3):
                wait_send(acc_send.at[rows_g, :], strip_send_sems.at[d_i, 0])
                wait_send(l_send.at[rows_g, :], strip_send_sems.at[d_i, 2])
                if with_b1:
                    wait_send(acc_send.at[rows_b1, :],
                              strip_send_sems.at[d_i, 1])
                    wait_send(l_send.at[rows_b1, :],
                              strip_send_sems.at[d_i, 3])

        @pl.when(my == 1)
        def _():
            drain_strips(True)
            wait_send(bcast_acc.at[rows_a, :], fwd_send_sems.at[0])
            wait_send(bcast_l, fwd_send_sems.at[1])

        @pl.when(my == 2)
        def _():
            drain_strips(False)

        @pl.when(my == 3)
        def _():
            drain_strips(False)
            wait_send(bcast_acc.at[rows_b, :], fwd_send_sems.at[0])

        def exit_barrier(second_barrier):
            for off in (1, 2, 3):
                pl.semaphore_signal(second_barrier, inc=1,
                                    device_id=((my + off) % N_DEV,),
                                    device_id_type=pl.DeviceIdType.MESH)
            pl.semaphore_wait(second_barrier, N_DEV - 1)

        pl.run_scoped(exit_barrier,
                      second_barrier=pltpu.SemaphoreType.REGULAR)

    out = pl.pallas_call(
        body,
        out_shape=jax.ShapeDtypeStruct((SQ, D), jnp.float32),
        in_specs=[pl.BlockSpec(memory_space=pltpu.VMEM)] * 5,
        out_specs=pl.BlockSpec(memory_space=pltpu.VMEM),
        scratch_shapes=[
            pltpu.VMEM((SQ, D), jnp.bfloat16),
            pltpu.VMEM((SQ, HQ), jnp.float32),
            pltpu.VMEM((SQ, D), jnp.bfloat16),
            pltpu.VMEM((SQ, HQ), jnp.float32),
            pltpu.VMEM((N_DEV, G + NB1, D), jnp.bfloat16),
            pltpu.VMEM((N_DEV, G + NB1, HQ), jnp.float32),
            pltpu.VMEM((SQ, D), jnp.float32),
            pltpu.VMEM((SQ, HQ), jnp.float32),
            pltpu.SemaphoreType.DMA((6,)),
            pltpu.SemaphoreType.DMA((3,)),
            pltpu.SemaphoreType.DMA((2,)),
            pltpu.SemaphoreType.DMA((3, 4)),
            pltpu.SemaphoreType.DMA((N_DEV, 4)),
        ],
        compiler_params=pltpu.CompilerParams(collective_id=0),
    )(x2, Wq, K2, V2, Wo)
    return out.reshape(1, SQ, D)


# device time: 57972 ns/iter; 2.0807x vs baseline; 1.2219x over previous
import jax
import jax.numpy as jnp
from jax import lax
from jax.experimental import pallas as pl
from jax.experimental.pallas import tpu as pltpu

N_DEV = 4
SQ = 1024
SKV_LOC = 1024
HQ = 8
DH = 128
D = 1024
SCALE = 0.08838834764831843
G = 32
B1 = 896
NB1 = SQ - B1
BR = 128
BW = 384
NBLK = SQ // BR

F32 = jnp.float32
BF16 = jnp.bfloat16


def _band_masks():
    r = lax.broadcasted_iota(jnp.int32, (BR, BW), 0)
    c = lax.broadcasted_iota(jnp.int32, (BR, BW), 1)
    m0 = ((jnp.abs(r - c) <= 128) | (c < 32)) & (r >= 32)
    m1 = ((c >= r) & (c <= r + 256)) | (c < 32)
    mg = (c >= r) & (c <= r + 256)
    m7 = c >= r + 128
    return m0, m1, mg, m7


def kernel(x, Wq, K_ext, V_ext, Wo):
    x2 = x.reshape(SQ, D)
    K2 = K_ext.reshape(SKV_LOC, HQ * DH)
    V2 = V_ext.reshape(SKV_LOC, HQ * DH)

    def body(x_ref, wq_ref, k_ref, v_ref, wo_ref, out_ref,
             acc_send, l_send, bcast_acc, bcast_l, strip_acc, strip_l,
             acc_tot, l_tot,
             bc_send_sems, bc_recv_sems, bcl_send_sems, bcl_recv_sems,
             fwd_send_sems, strip_send_sems, strip_recv_sems):
        my = lax.axis_index("i")

        def rcopy(src, dst, ssem, rsem, dev):
            return pltpu.make_async_remote_copy(
                src_ref=src, dst_ref=dst, send_sem=ssem, recv_sem=rsem,
                device_id=(dev,), device_id_type=pl.DeviceIdType.MESH)

        def wait_recv(dst, rsem):
            rcopy(dst, dst, rsem, rsem, 0).wait_recv()

        def wait_send(src, ssem):
            rcopy(src, src, ssem, ssem, 0).wait_send()

        def mm(a, b):
            return jnp.dot(a, b, preferred_element_type=F32)

        def mmT(a, b):
            return lax.dot_general(a, b, (((1,), (1,)), ((), ())),
                                   preferred_element_type=F32)

        barrier_sem = pltpu.get_barrier_semaphore()
        for off in (1, 2, 3):
            pl.semaphore_signal(barrier_sem, inc=1,
                                device_id=((my + off) % N_DEV,),
                                device_id_type=pl.DeviceIdType.MESH)
        pl.semaphore_wait(barrier_sem, N_DEV - 1)

        rows_g = pl.ds(0, G)
        rows_b1 = pl.ds(B1, NB1)
        rows_sg = pl.ds(0, G)
        rows_sb1 = pl.ds(G, NB1)

        xb = x_ref[...].astype(BF16)
        wqb = wq_ref[...].astype(BF16)
        q = mm(xb, wqb)
        q = (q * SCALE).astype(BF16)

        @pl.when(my == 0)
        def _():
            m0, m1, mg, m7 = _band_masks()
            cglob = lax.broadcasted_iota(jnp.int32, (BR, BR), 1) < 32
            for h in range(HQ):
                sl = slice(h * DH, (h + 1) * DH)
                kh = k_ref[:, sl].astype(BF16)
                vh = v_ref[:, sl].astype(BF16)
                qh = q[:, sl]
                for b in range(NBLK):
                    w0 = min(max(0, BR * b - BR), SKV_LOC - BW)
                    rows = pl.ds(BR * b, BR)
                    mask = {0: m0, 1: m1, NBLK - 1: m7}.get(b, mg)
                    s_b = mmT(qh[BR * b:BR * b + BR], kh[w0:w0 + BW])
                    w_b = jnp.where(mask, jnp.exp(s_b), 0.0)
                    lb = jnp.sum(w_b, axis=1, keepdims=True)
                    accb = mm(w_b.astype(BF16), vh[w0:w0 + BW])
                    if b >= 2:
                        s_s = mmT(qh[BR * b:BR * b + BR], kh[0:BR])
                        w_s = jnp.where(cglob, jnp.exp(s_s), 0.0)
                        lb = lb + jnp.sum(w_s, axis=1, keepdims=True)
                        accb = accb + mm(w_s.astype(BF16), vh[0:BR])
                    acc_send[rows, sl] = accb.astype(BF16)
                    l_send[rows, h:h + 1] = lb
                w_g = jnp.exp(mmT(qh[0:G], kh))
                l_send[rows_g, h:h + 1] = jnp.sum(w_g, axis=1, keepdims=True)
                acc_send[rows_g, sl] = mm(w_g.astype(BF16), vh).astype(BF16)
                rcopy(acc_send.at[:, sl], bcast_acc.at[:, sl],
                      bc_send_sems.at[0, h], bc_recv_sems.at[h], 1).start()
                rcopy(acc_send.at[:, sl], bcast_acc.at[:, sl],
                      bc_send_sems.at[1, h], bc_recv_sems.at[h], 3).start()
            rcopy(l_send, bcast_l,
                  bcl_send_sems.at[0], bcl_recv_sems.at[0], 1).start()
            rcopy(l_send, bcast_l,
                  bcl_send_sems.at[1], bcl_recv_sems.at[0], 3).start()

        def strip_compute_send(src_id, dsts, with_b1):
            for h in range(HQ):
                sl = slice(h * DH, (h + 1) * DH)
                kh = k_ref[:, sl].astype(BF16)
                vh = v_ref[:, sl].astype(BF16)
                w_g = jnp.exp(mmT(q[0:G, sl], kh))
                l_send[rows_g, h:h + 1] = jnp.sum(w_g, axis=1, keepdims=True)
                acc_send[rows_g, sl] = mm(w_g.astype(BF16), vh).astype(BF16)
                if with_b1:
                    r = lax.broadcasted_iota(jnp.int32, (NB1, BR), 0)
                    c = lax.broadcasted_iota(jnp.int32, (NB1, BR), 1)
                    s_b = mmT(q[B1:SQ, sl], kh[0:BR])
                    w_b = jnp.where(c <= r, jnp.exp(s_b), 0.0)
                    l_send[rows_b1, h:h + 1] = jnp.sum(w_b, axis=1,
                                                       keepdims=True)
                    acc_send[rows_b1, sl] = mm(w_b.astype(BF16),
                                               vh[0:BR]).astype(BF16)
            for d_i, dst in enumerate(dsts):
                rcopy(acc_send.at[rows_g, :], strip_acc.at[src_id, rows_sg, :],
                      strip_send_sems.at[d_i, 0],
                      strip_recv_sems.at[src_id, 0], dst).start()
                rcopy(l_send.at[rows_g, :], strip_l.at[src_id, rows_sg, :],
                      strip_send_sems.at[d_i, 2],
                      strip_recv_sems.at[src_id, 2], dst).start()
                if with_b1:
                    rcopy(acc_send.at[rows_b1, :],
                          strip_acc.at[src_id, rows_sb1, :],
                          strip_send_sems.at[d_i, 1],
                          strip_recv_sems.at[src_id, 1], dst).start()
                    rcopy(l_send.at[rows_b1, :],
                          strip_l.at[src_id, rows_sb1, :],
                          strip_send_sems.at[d_i, 3],
                          strip_recv_sems.at[src_id, 3], dst).start()

        @pl.when(my == 1)
        def _():
            strip_compute_send(1, (0, 2, 3), True)
            for h in range(HQ):
                sl = slice(h * DH, (h + 1) * DH)
                wait_recv(bcast_acc.at[:, sl], bc_recv_sems.at[h])
                if h % 2 == 0:
                    rcopy(bcast_acc.at[:, sl], bcast_acc.at[:, sl],
                          fwd_send_sems.at[h], bc_recv_sems.at[h], 2).start()
            wait_recv(bcast_l, bcl_recv_sems.at[0])
            rcopy(bcast_l, bcast_l,
                  fwd_send_sems.at[8], bcl_recv_sems.at[0], 2).start()

        @pl.when(my == 2)
        def _():
            strip_compute_send(2, (0, 1, 3), False)
            for h in range(HQ):
                wait_recv(bcast_acc.at[:, h * DH:(h + 1) * DH],
                          bc_recv_sems.at[h])
            wait_recv(bcast_l, bcl_recv_sems.at[0])

        @pl.when(my == 3)
        def _():
            strip_compute_send(3, (0, 1, 2), False)
            for h in range(HQ):
                sl = slice(h * DH, (h + 1) * DH)
                wait_recv(bcast_acc.at[:, sl], bc_recv_sems.at[h])
                if h % 2 == 1:
                    rcopy(bcast_acc.at[:, sl], bcast_acc.at[:, sl],
                          fwd_send_sems.at[h], bc_recv_sems.at[h], 2).start()
            wait_recv(bcast_l, bcl_recv_sems.at[0])

        @pl.when(my == 0)
        def _():
            acc_tot[...] = acc_send[...].astype(F32)
            l_tot[...] = l_send[...]

        @pl.when(my != 0)
        def _():
            acc_tot[...] = bcast_acc[...].astype(F32)
            l_tot[...] = bcast_l[...]

        for s_id in (1, 2, 3):
            @pl.when(my == s_id)
            def _(s_id=s_id):
                acc_tot[rows_g, :] = (acc_tot[rows_g, :]
                                      + acc_send[rows_g, :].astype(F32))
                l_tot[rows_g, :] = l_tot[rows_g, :] + l_send[rows_g, :]
                if s_id == 1:
                    acc_tot[rows_b1, :] = (acc_tot[rows_b1, :]
                                           + acc_send[rows_b1, :].astype(F32))
                    l_tot[rows_b1, :] = l_tot[rows_b1, :] + l_send[rows_b1, :]

            @pl.when(my != s_id)
            def _(s_id=s_id):
                wait_recv(strip_acc.at[s_id, rows_sg, :],
                          strip_recv_sems.at[s_id, 0])
                wait_recv(strip_l.at[s_id, rows_sg, :],
                          strip_recv_sems.at[s_id, 2])
                acc_tot[rows_g, :] = (
                    acc_tot[rows_g, :]
                    + strip_acc[s_id, rows_sg, :].astype(F32))
                l_tot[rows_g, :] = l_tot[rows_g, :] + strip_l[s_id, rows_sg, :]
                if s_id == 1:
                    wait_recv(strip_acc.at[s_id, rows_sb1, :],
                              strip_recv_sems.at[s_id, 1])
                    wait_recv(strip_l.at[s_id, rows_sb1, :],
                              strip_recv_sems.at[s_id, 3])
                    acc_tot[rows_b1, :] = (
                        acc_tot[rows_b1, :]
                        + strip_acc[s_id, rows_sb1, :].astype(F32))
                    l_tot[rows_b1, :] = (
                        l_tot[rows_b1, :] + strip_l[s_id, rows_sb1, :])

        accf = acc_tot[...]
        lf = l_tot[...]
        ctx = jnp.concatenate(
            [accf[:, h * DH:(h + 1) * DH] / lf[:, h:h + 1] for h in range(HQ)],
            axis=1,
        ).astype(BF16)
        out_ref[...] = mm(ctx, wo_ref[...].astype(BF16))

        @pl.when(my == 0)
        def _():
            for h in range(HQ):
                sl = slice(h * DH, (h + 1) * DH)
                wait_send(acc_send.at[:, sl], bc_send_sems.at[0, h])
                wait_send(acc_send.at[:, sl], bc_send_sems.at[1, h])
            wait_send(l_send, bcl_send_sems.at[0])
            wait_send(l_send, bcl_send_sems.at[1])

        def drain_strips(with_b1):
            for d_i in range(3):
                wait_send(acc_send.at[rows_g, :], strip_send_sems.at[d_i, 0])
                wait_send(l_send.at[rows_g, :], strip_send_sems.at[d_i, 2])
                if with_b1:
                    wait_send(acc_send.at[rows_b1, :],
                              strip_send_sems.at[d_i, 1])
                    wait_send(l_send.at[rows_b1, :],
                              strip_send_sems.at[d_i, 3])

        @pl.when(my == 1)
        def _():
            drain_strips(True)
            for h in (0, 2, 4, 6):
                wait_send(bcast_acc.at[:, h * DH:(h + 1) * DH],
                          fwd_send_sems.at[h])
            wait_send(bcast_l, fwd_send_sems.at[8])

        @pl.when(my == 2)
        def _():
            drain_strips(False)

        @pl.when(my == 3)
        def _():
            drain_strips(False)
            for h in (1, 3, 5, 7):
                wait_send(bcast_acc.at[:, h * DH:(h + 1) * DH],
                          fwd_send_sems.at[h])

        def exit_barrier(second_barrier):
            for off in (1, 2, 3):
                pl.semaphore_signal(second_barrier, inc=1,
                                    device_id=((my + off) % N_DEV,),
                                    device_id_type=pl.DeviceIdType.MESH)
            pl.semaphore_wait(second_barrier, N_DEV - 1)

        pl.run_scoped(exit_barrier,
                      second_barrier=pltpu.SemaphoreType.REGULAR)

    out = pl.pallas_call(
        body,
        out_shape=jax.ShapeDtypeStruct((SQ, D), jnp.float32),
        in_specs=[pl.BlockSpec(memory_space=pltpu.VMEM)] * 5,
        out_specs=pl.BlockSpec(memory_space=pltpu.VMEM),
        scratch_shapes=[
            pltpu.VMEM((SQ, D), BF16),
            pltpu.VMEM((SQ, HQ), F32),
            pltpu.VMEM((SQ, D), BF16),
            pltpu.VMEM((SQ, HQ), F32),
            pltpu.VMEM((N_DEV, G + NB1, D), BF16),
            pltpu.VMEM((N_DEV, G + NB1, HQ), F32),
            pltpu.VMEM((SQ, D), F32),
            pltpu.VMEM((SQ, HQ), F32),
            pltpu.SemaphoreType.DMA((2, HQ)),
            pltpu.SemaphoreType.DMA((HQ,)),
            pltpu.SemaphoreType.DMA((2,)),
            pltpu.SemaphoreType.DMA((1,)),
            pltpu.SemaphoreType.DMA((HQ + 1,)),
            pltpu.SemaphoreType.DMA((3, 4)),
            pltpu.SemaphoreType.DMA((N_DEV, 4)),
        ],
        compiler_params=pltpu.CompilerParams(collective_id=0),
    )(x2, Wq, K2, V2, Wo)
    return out.reshape(1, SQ, D)
